# baseline (device time: 12844 ns/iter reference)
import jax
import jax.numpy as jnp
from jax import lax
from jax.experimental import pallas as pl
from jax.experimental.pallas import tpu as pltpu

N_DEV = 16
N_TOK = 256
D_IN = 128
D_OUT = 256
ROWS = N_TOK // N_DEV
N_EXP = 32
E_LOCAL = 2


def kernel(x, router_W, route_idx, expert_W):
    def body(x_ref, rw_ref, idx_ref, ew_ref, out_ref,
             partial_ref, recv_ref, send_sems, recv_sems):
        my = lax.axis_index("i")

        barrier_sem = pltpu.get_barrier_semaphore()
        for k in range(1, N_DEV):
            pl.semaphore_signal(
                barrier_sem, inc=1,
                device_id=((my + k) % N_DEV,),
                device_id_type=pl.DeviceIdType.MESH,
            )
        pl.semaphore_wait(barrier_sem, N_DEV - 1)

        xf = x_ref[:, :]
        scores = jnp.dot(xf, rw_ref[:, :], preferred_element_type=jnp.float32)
        s_max = jnp.max(scores, axis=-1, keepdims=True)
        exps = jnp.exp(scores - s_max)

        idx = idx_ref[:, :]
        i0 = idx[:, 0:1]
        i1 = idx[:, 1:2]
        iota = lax.broadcasted_iota(jnp.int32, (N_TOK, N_EXP), 1)
        g0 = jnp.sum(jnp.where(iota == i0, exps, 0.0), axis=-1, keepdims=True)
        g1 = jnp.sum(jnp.where(iota == i1, exps, 0.0), axis=-1, keepdims=True)
        denom = g0 + g1

        xb = xf.astype(jnp.bfloat16)
        acc = jnp.zeros((N_TOK, D_OUT), jnp.float32)
        for e in range(E_LOCAL):
            le = my * E_LOCAL + e
            gate = (jnp.where(i0 == le, g0, 0.0)
                    + jnp.where(i1 == le, g1, 0.0)) / denom
            y = jnp.dot(xb, ew_ref[e].astype(jnp.bfloat16),
                        preferred_element_type=jnp.float32)
            acc = acc + gate * y
        partial_ref[:, :] = acc

        rdmas = []
        for k in range(1, N_DEV):
            dst = (my + k) % N_DEV
            rdma = pltpu.make_async_remote_copy(
                src_ref=partial_ref.at[pl.ds(dst * ROWS, ROWS), :],
                dst_ref=recv_ref.at[k - 1],
                send_sem=send_sems.at[k - 1],
                recv_sem=recv_sems.at[k - 1],
                device_id=(dst,),
                device_id_type=pl.DeviceIdType.MESH,
            )
            rdma.start()
            rdmas.append(rdma)

        out = partial_ref[pl.ds(my * ROWS, ROWS), :]
        for k in range(1, N_DEV):
            rdmas[k - 1].wait_recv()
            out = out + recv_ref[k - 1]
        out_ref[:, :] = out

        for k in range(1, N_DEV):
            rdmas[k - 1].wait_send()

    return pl.pallas_call(
        body,
        out_shape=jax.ShapeDtypeStruct((ROWS, D_OUT), jnp.float32),
        in_specs=[
            pl.BlockSpec(memory_space=pltpu.VMEM),
            pl.BlockSpec(memory_space=pltpu.VMEM),
            pl.BlockSpec(memory_space=pltpu.VMEM),
            pl.BlockSpec(memory_space=pltpu.VMEM),
        ],
        out_specs=pl.BlockSpec(memory_space=pltpu.VMEM),
        scratch_shapes=[
            pltpu.VMEM((N_TOK, D_OUT), jnp.float32),
            pltpu.VMEM((N_DEV - 1, ROWS, D_OUT), jnp.float32),
            pltpu.SemaphoreType.DMA((N_DEV - 1,)),
            pltpu.SemaphoreType.DMA((N_DEV - 1,)),
        ],
        compiler_params=pltpu.CompilerParams(collective_id=0),
    )(x, router_W, route_idx, expert_W)


# device time: 11422 ns/iter; 1.1245x vs baseline; 1.1245x over previous
import jax
import jax.numpy as jnp
from jax import lax
from jax.experimental import pallas as pl
from jax.experimental.pallas import tpu as pltpu

N_DEV = 16
N_TOK = 256
D_IN = 128
D_OUT = 256
ROWS = N_TOK // N_DEV
N_EXP = 32
E_LOCAL = 2


def kernel(x, router_W, route_idx, expert_W):
    def body(x_ref, rw_ref, idx_ref, ew_ref, out_ref,
             partial_ref, recv_ref, send_sems, recv_sems):
        my = lax.axis_index("i")

        barrier_sem = pltpu.get_barrier_semaphore()
        for k in range(1, N_DEV):
            pl.semaphore_signal(
                barrier_sem, inc=1,
                device_id=((my + k) % N_DEV,),
                device_id_type=pl.DeviceIdType.MESH,
            )

        xf = x_ref[:, :]
        scores = jnp.dot(xf, rw_ref[:, :], preferred_element_type=jnp.float32)
        s_max = jnp.max(scores, axis=-1, keepdims=True)
        exps = jnp.exp(scores - s_max)

        idx = idx_ref[:, :]
        i0 = idx[:, 0:1]
        i1 = idx[:, 1:2]
        iota = lax.broadcasted_iota(jnp.int32, (N_TOK, N_EXP), 1)
        g0 = jnp.sum(jnp.where(iota == i0, exps, 0.0), axis=-1, keepdims=True)
        g1 = jnp.sum(jnp.where(iota == i1, exps, 0.0), axis=-1, keepdims=True)
        denom = g0 + g1

        xb = xf.astype(jnp.bfloat16)
        acc = jnp.zeros((N_TOK, D_OUT), jnp.float32)
        for e in range(E_LOCAL):
            le = my * E_LOCAL + e
            gate = (jnp.where(i0 == le, g0, 0.0)
                    + jnp.where(i1 == le, g1, 0.0)) / denom
            y = jnp.dot(xb, ew_ref[e].astype(jnp.bfloat16),
                        preferred_element_type=jnp.float32)
            acc = acc + gate * y
        partial_ref[:, :] = acc.astype(jnp.bfloat16)

        pl.semaphore_wait(barrier_sem, N_DEV - 1)

        rdmas = []
        for k in range(1, N_DEV):
            dst = (my + k) % N_DEV
            rdma = pltpu.make_async_remote_copy(
                src_ref=partial_ref.at[pl.ds(dst * ROWS, ROWS), :],
                dst_ref=recv_ref.at[k - 1],
                send_sem=send_sems.at[k - 1],
                recv_sem=recv_sems.at[k - 1],
                device_id=(dst,),
                device_id_type=pl.DeviceIdType.MESH,
            )
            rdma.start()
            rdmas.append(rdma)

        out = partial_ref[pl.ds(my * ROWS, ROWS), :].astype(jnp.float32)
        for k in range(1, N_DEV):
            rdmas[k - 1].wait_recv()
            out = out + recv_ref[k - 1].astype(jnp.float32)
        out_ref[:, :] = out

        for k in range(1, N_DEV):
            rdmas[k - 1].wait_send()

    return pl.pallas_call(
        body,
        out_shape=jax.ShapeDtypeStruct((ROWS, D_OUT), jnp.float32),
        in_specs=[
            pl.BlockSpec(memory_space=pltpu.VMEM),
            pl.BlockSpec(memory_space=pltpu.VMEM),
            pl.BlockSpec(memory_space=pltpu.VMEM),
            pl.BlockSpec(memory_space=pltpu.VMEM),
        ],
        out_specs=pl.BlockSpec(memory_space=pltpu.VMEM),
        scratch_shapes=[
            pltpu.VMEM((N_TOK, D_OUT), jnp.bfloat16),
            pltpu.VMEM((N_DEV - 1, ROWS, D_OUT), jnp.bfloat16),
            pltpu.SemaphoreType.DMA((N_DEV - 1,)),
            pltpu.SemaphoreType.DMA((N_DEV - 1,)),
        ],
        compiler_params=pltpu.CompilerParams(collective_id=0),
    )(x, router_W, route_idx, expert_W)


# device time: 11391 ns/iter; 1.1276x vs baseline; 1.0027x over previous
import jax
import jax.numpy as jnp
from jax import lax
from jax.experimental import pallas as pl
from jax.experimental.pallas import tpu as pltpu

N_DEV = 16
N_TOK = 256
D_IN = 128
D_OUT = 256
ROWS = N_TOK // N_DEV
N_EXP = 32
E_LOCAL = 2


def kernel(x, router_W, route_idx, expert_W):
    def body(x_ref, rw_ref, idx_ref, ew_ref, out_ref,
             partial_ref, recv_ref, send_sems, recv_sems):
        my = lax.axis_index("i")

        barrier_sem = pltpu.get_barrier_semaphore()
        for k in range(1, N_DEV):
            pl.semaphore_signal(
                barrier_sem, inc=1,
                device_id=((my + k) % N_DEV,),
                device_id_type=pl.DeviceIdType.MESH,
            )

        xf = x_ref[:, :]
        scores = jnp.dot(xf, rw_ref[:, :], preferred_element_type=jnp.float32)
        s_max = jnp.max(scores, axis=-1, keepdims=True)
        exps = jnp.exp(scores - s_max)

        idx = idx_ref[:, :]
        i0 = idx[:, 0:1]
        i1 = idx[:, 1:2]
        iota = lax.broadcasted_iota(jnp.int32, (N_TOK, N_EXP), 1)
        g0 = jnp.sum(jnp.where(iota == i0, exps, 0.0), axis=-1, keepdims=True)
        g1 = jnp.sum(jnp.where(iota == i1, exps, 0.0), axis=-1, keepdims=True)
        denom = g0 + g1

        gx = []
        for e in range(E_LOCAL):
            le = my * E_LOCAL + e
            gate = (jnp.where(i0 == le, g0, 0.0)
                    + jnp.where(i1 == le, g1, 0.0)) / denom
            gx.append(xf * gate)
        gxb = jnp.concatenate(gx, axis=1).astype(jnp.bfloat16)
        wb = ew_ref[:, :, :].reshape(E_LOCAL * D_IN, D_OUT).astype(jnp.bfloat16)
        acc = jnp.dot(gxb, wb, preferred_element_type=jnp.float32)
        partial_ref[:, :] = acc.astype(jnp.bfloat16)

        pl.semaphore_wait(barrier_sem, N_DEV - 1)

        rdmas = []
        for k in range(1, N_DEV):
            dst = (my + k) % N_DEV
            rdma = pltpu.make_async_remote_copy(
                src_ref=partial_ref.at[pl.ds(dst * ROWS, ROWS), :],
                dst_ref=recv_ref.at[k - 1],
                send_sem=send_sems.at[k - 1],
                recv_sem=recv_sems.at[k - 1],
                device_id=(dst,),
                device_id_type=pl.DeviceIdType.MESH,
            )
            rdma.start()
            rdmas.append(rdma)

        out = partial_ref[pl.ds(my * ROWS, ROWS), :].astype(jnp.float32)
        for k in range(1, N_DEV):
            rdmas[k - 1].wait_recv()
            out = out + recv_ref[k - 1].astype(jnp.float32)
        out_ref[:, :] = out

        for k in range(1, N_DEV):
            rdmas[k - 1].wait_send()

    return pl.pallas_call(
        body,
        out_shape=jax.ShapeDtypeStruct((ROWS, D_OUT), jnp.float32),
        in_specs=[
            pl.BlockSpec(memory_space=pltpu.VMEM),
            pl.BlockSpec(memory_space=pltpu.VMEM),
            pl.BlockSpec(memory_space=pltpu.VMEM),
            pl.BlockSpec(memory_space=pltpu.VMEM),
        ],
        out_specs=pl.BlockSpec(memory_space=pltpu.VMEM),
        scratch_shapes=[
            pltpu.VMEM((N_TOK, D_OUT), jnp.bfloat16),
            pltpu.VMEM((N_DEV - 1, ROWS, D_OUT), jnp.bfloat16),
            pltpu.SemaphoreType.DMA((N_DEV - 1,)),
            pltpu.SemaphoreType.DMA((N_DEV - 1,)),
        ],
        compiler_params=pltpu.CompilerParams(collective_id=0),
    )(x, router_W, route_idx, expert_W)
